# baseline (speedup 1.0000x reference)
import jax
import jax.numpy as jnp
from jax.experimental import pallas as pl
from jax.experimental.pallas import tpu as pltpu

_B = 1024
_NF = 100000
_D = 64
_H1 = 64
_H2 = 32
_BT = 8
_BB = _B // _BT
_KT = 4096
_NT = _NF // _KT
_REM = _NF - _NT * _KT
_NG = _NT + 1
_EPS = 1e-5

_TDOT = (((0,), (1,)), ((), ()))


def _acc_kernel(x_ref, e_ref, lw_ref, a_ref, q_ref):
    k = pl.program_id(1)

    @pl.when(k == 0)
    def _init():
        a_ref[...] = jnp.zeros_like(a_ref)
        q_ref[...] = jnp.zeros_like(q_ref)

    def _accumulate(x, e, lw):
        aug = jnp.concatenate([e, lw], axis=1)
        a_ref[...] += jax.lax.dot_general(
            aug, x, _TDOT, preferred_element_type=jnp.float32)
        q_ref[...] += jax.lax.dot_general(
            e * e, x * x, _TDOT, preferred_element_type=jnp.float32)

    @pl.when(k < _NT)
    def _full_tile():
        _accumulate(x_ref[...].astype(jnp.bfloat16),
                    e_ref[...].astype(jnp.bfloat16),
                    lw_ref[...].astype(jnp.bfloat16))

    @pl.when(k == _NT)
    def _tail_tile():
        lane = jax.lax.broadcasted_iota(jnp.int32, (1, _KT), 1)
        sub = jax.lax.broadcasted_iota(jnp.int32, (_KT, 1), 0)
        x = jnp.where(lane < _REM, x_ref[...], 0.0).astype(jnp.bfloat16)
        e = jnp.where(sub < _REM, e_ref[...], 0.0).astype(jnp.bfloat16)
        lw = jnp.where(sub < _REM, lw_ref[...], 0.0).astype(jnp.bfloat16)
        _accumulate(x, e, lw)


def _bn_t(v, g, b):
    mu = jnp.mean(v, axis=1, keepdims=True)
    var = jnp.mean(jnp.square(v - mu), axis=1, keepdims=True)
    return (v - mu) / jnp.sqrt(var + _EPS) * g + b


def _tail_kernel(a_ref, q_ref, lb_ref, g0_ref, b0_ref,
                 w1_ref, b1_ref, g1_ref, bb1_ref,
                 w2_ref, b2_ref, g2_ref, bb2_ref, hw_ref, out_ref):
    se = a_ref[:_D, :]
    lin = a_ref[_D:_D + 1, :]
    bi = 0.5 * (se * se - q_ref[...])
    z = _bn_t(bi, g0_ref[...], b0_ref[...])
    z = jnp.dot(w1_ref[...], z,
                preferred_element_type=jnp.float32) + b1_ref[...]
    z = jax.nn.relu(_bn_t(z, g1_ref[...], bb1_ref[...]))
    z = jnp.dot(w2_ref[...], z,
                preferred_element_type=jnp.float32) + b2_ref[...]
    z = jax.nn.relu(_bn_t(z, g2_ref[...], bb2_ref[...]))
    y = jnp.sum(z * hw_ref[...], axis=0, keepdims=True)
    out_ref[...] = y + lin + lb_ref[...]


def kernel(feature_values, feature_embed, lin_w, lin_b, bn0_g, bn0_b,
           W1, b1, bn1_g, bn1_b, W2, b2, bn2_g, bn2_b, h_w):
    acc_a, acc_q = pl.pallas_call(
        _acc_kernel,
        grid=(_BT, _NG),
        in_specs=[
            pl.BlockSpec((_BB, _KT), lambda b, k: (b, k)),
            pl.BlockSpec((_KT, _D), lambda b, k: (k, 0)),
            pl.BlockSpec((_KT, 1), lambda b, k: (k, 0)),
        ],
        out_specs=[
            pl.BlockSpec((_D + 1, _BB), lambda b, k: (0, b)),
            pl.BlockSpec((_D, _BB), lambda b, k: (0, b)),
        ],
        out_shape=[
            jax.ShapeDtypeStruct((_D + 1, _B), jnp.float32),
            jax.ShapeDtypeStruct((_D, _B), jnp.float32),
        ],
        compiler_params=pltpu.CompilerParams(
            dimension_semantics=("parallel", "arbitrary"),
        ),
    )(feature_values, feature_embed, lin_w.reshape(_NF, 1))

    out = pl.pallas_call(
        _tail_kernel,
        out_shape=jax.ShapeDtypeStruct((1, _B), jnp.float32),
    )(acc_a, acc_q,
      lin_b.reshape(1, 1), bn0_g.reshape(_D, 1), bn0_b.reshape(_D, 1),
      W1, b1.reshape(_H1, 1), bn1_g.reshape(_H1, 1), bn1_b.reshape(_H1, 1),
      W2, b2.reshape(_H2, 1), bn2_g.reshape(_H2, 1), bn2_b.reshape(_H2, 1),
      h_w.reshape(_H2, 1))
    return out.reshape(_B)


# speedup vs baseline: 1.2467x; 1.2467x over previous
import jax
import jax.numpy as jnp
from jax.experimental import pallas as pl
from jax.experimental.pallas import tpu as pltpu

_B = 1024
_NF = 100000
_D = 64
_H1 = 64
_H2 = 32
_BB = 64
_BT = _B // _BB
_KT = 25600
_NK = 4
_REM = _NF - (_NK - 1) * _KT
_EPS = 1e-5

_TDOT = (((0,), (1,)), ((), ()))


def _acc_kernel(x_ref, aug_ref, a_ref, q_ref, a_scr, q_scr):
    k = pl.program_id(0)
    b = pl.program_id(1)

    thresh = jnp.where(k < _NK - 1, _KT, _REM)
    lane = jax.lax.broadcasted_iota(jnp.int32, (1, _KT), 1)
    sub = jax.lax.broadcasted_iota(jnp.int32, (_KT, 1), 0)
    x = jnp.where(lane < thresh, x_ref[...], 0.0)
    aug = jnp.where(sub < thresh, aug_ref[...], 0)

    xb = x.astype(jnp.bfloat16)
    e = aug[:, :_D]
    pa = jax.lax.dot_general(
        aug, xb, _TDOT, preferred_element_type=jnp.float32)
    pq = jax.lax.dot_general(
        e * e, xb * xb, _TDOT, preferred_element_type=jnp.float32)

    @pl.when(k == 0)
    def _first_step():
        a_scr[b] = pa
        q_scr[b] = pq

    @pl.when(jnp.logical_and(k > 0, k < _NK - 1))
    def _middle_step():
        a_scr[b] += pa
        q_scr[b] += pq

    @pl.when(k == _NK - 1)
    def _last_step():
        a_ref[0] = a_scr[b] + pa
        q_ref[0] = q_scr[b] + pq


def _bn_t(v, g, b):
    mu = jnp.mean(v, axis=1, keepdims=True)
    var = jnp.mean(jnp.square(v - mu), axis=1, keepdims=True)
    return (v - mu) / jnp.sqrt(var + _EPS) * g + b


def _tail_kernel(a_ref, q_ref, lb_ref, g0_ref, b0_ref,
                 w1_ref, b1_ref, g1_ref, bb1_ref,
                 w2_ref, b2_ref, g2_ref, bb2_ref, hw_ref, out_ref):
    se = a_ref[:_D, :]
    lin = a_ref[_D:_D + 1, :]
    bi = 0.5 * (se * se - q_ref[...])
    z = _bn_t(bi, g0_ref[...], b0_ref[...])
    z = jnp.dot(w1_ref[...], z,
                preferred_element_type=jnp.float32) + b1_ref[...]
    z = jax.nn.relu(_bn_t(z, g1_ref[...], bb1_ref[...]))
    z = jnp.dot(w2_ref[...], z,
                preferred_element_type=jnp.float32) + b2_ref[...]
    z = jax.nn.relu(_bn_t(z, g2_ref[...], bb2_ref[...]))
    y = jnp.sum(z * hw_ref[...], axis=0, keepdims=True)
    out_ref[...] = y + lin + lb_ref[...]


def kernel(feature_values, feature_embed, lin_w, lin_b, bn0_g, bn0_b,
           W1, b1, bn1_g, bn1_b, W2, b2, bn2_g, bn2_b, h_w):
    aug_bf = jnp.concatenate(
        [feature_embed, lin_w.reshape(_NF, 1)], axis=1).astype(jnp.bfloat16)

    acc_a, acc_q = pl.pallas_call(
        _acc_kernel,
        grid=(_NK, _BT),
        in_specs=[
            pl.BlockSpec((_BB, _KT), lambda k, b: (b, k)),
            pl.BlockSpec((_KT, _D + 1), lambda k, b: (k, 0)),
        ],
        out_specs=[
            pl.BlockSpec((1, _D + 1, _BB), lambda k, b: (b, 0, 0)),
            pl.BlockSpec((1, _D, _BB), lambda k, b: (b, 0, 0)),
        ],
        out_shape=[
            jax.ShapeDtypeStruct((_BT, _D + 1, _BB), jnp.float32),
            jax.ShapeDtypeStruct((_BT, _D, _BB), jnp.float32),
        ],
        scratch_shapes=[
            pltpu.VMEM((_BT, _D + 1, _BB), jnp.float32),
            pltpu.VMEM((_BT, _D, _BB), jnp.float32),
        ],
        compiler_params=pltpu.CompilerParams(
            dimension_semantics=("arbitrary", "arbitrary"),
        ),
    )(feature_values, aug_bf)

    acc_a = jnp.transpose(acc_a, (1, 0, 2)).reshape(_D + 1, _B)
    acc_q = jnp.transpose(acc_q, (1, 0, 2)).reshape(_D, _B)

    out = pl.pallas_call(
        _tail_kernel,
        out_shape=jax.ShapeDtypeStruct((1, _B), jnp.float32),
    )(acc_a, acc_q,
      lin_b.reshape(1, 1), bn0_g.reshape(_D, 1), bn0_b.reshape(_D, 1),
      W1, b1.reshape(_H1, 1), bn1_g.reshape(_H1, 1), bn1_b.reshape(_H1, 1),
      W2, b2.reshape(_H2, 1), bn2_g.reshape(_H2, 1), bn2_b.reshape(_H2, 1),
      h_w.reshape(_H2, 1))
    return out.reshape(_B)


# speedup vs baseline: 1.3807x; 1.1074x over previous
import jax
import jax.numpy as jnp
from jax.experimental import pallas as pl
from jax.experimental.pallas import tpu as pltpu

_B = 1024
_NF = 100000
_D = 64
_H1 = 64
_H2 = 32
_KT = 2048
_NT = _NF // _KT
_REM = _NF - _NT * _KT
_NG = _NT + 1
_NBUF = 4
_EPS = 1e-5

_TDOT = (((0,), (1,)), ((), ()))


def _acc_kernel(x_hbm, e_ref, lw_ref, a_ref, q_ref, xbuf, xtail, sem):
    k = pl.program_id(0)

    def _start(t):
        @pl.when(t < _NT)
        def _full_copy():
            pltpu.make_async_copy(
                x_hbm.at[:, pl.ds(t * _KT, _KT)],
                xbuf.at[jax.lax.rem(t, _NBUF)],
                sem.at[jax.lax.rem(t, _NBUF)]).start()

    @pl.when(k == 0)
    def _prologue():
        a_ref[...] = jnp.zeros_like(a_ref)
        q_ref[...] = jnp.zeros_like(q_ref)
        for t in range(min(_NBUF, _NT)):
            _start(jnp.int32(t))
        pltpu.make_async_copy(
            x_hbm.at[:, pl.ds(_NT * _KT, _REM)], xtail, sem.at[_NBUF]).start()

    slot = jax.lax.rem(k, _NBUF)

    @pl.when(k < _NT)
    def _wait_full():
        pltpu.make_async_copy(
            x_hbm.at[:, pl.ds(k * _KT, _KT)],
            xbuf.at[slot], sem.at[slot]).wait()

    @pl.when(k == _NT)
    def _wait_tail():
        pltpu.make_async_copy(
            x_hbm.at[:, pl.ds(_NT * _KT, _REM)], xtail, sem.at[_NBUF]).wait()

    def _accumulate(x, e, lw):
        aug = jnp.concatenate([e, lw], axis=1)
        a_ref[...] += jax.lax.dot_general(
            aug, x, _TDOT, preferred_element_type=jnp.float32)
        q_ref[...] += jax.lax.dot_general(
            e * e, x * x, _TDOT, preferred_element_type=jnp.float32)

    @pl.when(k < _NT)
    def _full_tile():
        _accumulate(xbuf[slot].astype(jnp.bfloat16),
                    e_ref[...].astype(jnp.bfloat16),
                    lw_ref[...].astype(jnp.bfloat16))

    @pl.when(k == _NT)
    def _tail_tile():
        _accumulate(xtail[...].astype(jnp.bfloat16),
                    e_ref[pl.ds(0, _REM), :].astype(jnp.bfloat16),
                    lw_ref[pl.ds(0, _REM), :].astype(jnp.bfloat16))

    _start(k + _NBUF)


def _bn_t(v, g, b):
    mu = jnp.mean(v, axis=1, keepdims=True)
    var = jnp.mean(jnp.square(v - mu), axis=1, keepdims=True)
    return (v - mu) / jnp.sqrt(var + _EPS) * g + b


def _tail_kernel(a_ref, q_ref, lb_ref, g0_ref, b0_ref,
                 w1_ref, b1_ref, g1_ref, bb1_ref,
                 w2_ref, b2_ref, g2_ref, bb2_ref, hw_ref, out_ref):
    se = a_ref[:_D, :]
    lin = a_ref[_D:_D + 1, :]
    bi = 0.5 * (se * se - q_ref[...])
    z = _bn_t(bi, g0_ref[...], b0_ref[...])
    z = jnp.dot(w1_ref[...], z,
                preferred_element_type=jnp.float32) + b1_ref[...]
    z = jax.nn.relu(_bn_t(z, g1_ref[...], bb1_ref[...]))
    z = jnp.dot(w2_ref[...], z,
                preferred_element_type=jnp.float32) + b2_ref[...]
    z = jax.nn.relu(_bn_t(z, g2_ref[...], bb2_ref[...]))
    y = jnp.sum(z * hw_ref[...], axis=0, keepdims=True)
    out_ref[...] = y + lin + lb_ref[...]


def kernel(feature_values, feature_embed, lin_w, lin_b, bn0_g, bn0_b,
           W1, b1, bn1_g, bn1_b, W2, b2, bn2_g, bn2_b, h_w):
    acc_a, acc_q = pl.pallas_call(
        _acc_kernel,
        grid=(_NG,),
        in_specs=[
            pl.BlockSpec(memory_space=pltpu.MemorySpace.HBM),
            pl.BlockSpec((_KT, _D), lambda k: (k, 0)),
            pl.BlockSpec((_KT, 1), lambda k: (k, 0)),
        ],
        out_specs=[
            pl.BlockSpec((_D + 1, _B), lambda k: (0, 0)),
            pl.BlockSpec((_D, _B), lambda k: (0, 0)),
        ],
        out_shape=[
            jax.ShapeDtypeStruct((_D + 1, _B), jnp.float32),
            jax.ShapeDtypeStruct((_D, _B), jnp.float32),
        ],
        scratch_shapes=[
            pltpu.VMEM((_NBUF, _B, _KT), jnp.float32),
            pltpu.VMEM((_B, _REM), jnp.float32),
            pltpu.SemaphoreType.DMA((_NBUF + 1,)),
        ],
        compiler_params=pltpu.CompilerParams(
            dimension_semantics=("arbitrary",),
        ),
    )(feature_values, feature_embed, lin_w.reshape(_NF, 1))

    out = pl.pallas_call(
        _tail_kernel,
        out_shape=jax.ShapeDtypeStruct((1, _B), jnp.float32),
    )(acc_a, acc_q,
      lin_b.reshape(1, 1), bn0_g.reshape(_D, 1), bn0_b.reshape(_D, 1),
      W1, b1.reshape(_H1, 1), bn1_g.reshape(_H1, 1), bn1_b.reshape(_H1, 1),
      W2, b2.reshape(_H2, 1), bn2_g.reshape(_H2, 1), bn2_b.reshape(_H2, 1),
      h_w.reshape(_H2, 1))
    return out.reshape(_B)
